# baseline (speedup 1.0000x reference)
import jax
import jax.numpy as jnp
from jax.experimental import pallas as pl
from jax.experimental.pallas import tpu as pltpu

E = 8
K = 2
R = 16
ALPHA = 16
SCALING = ALPHA / R

TM = 512
ER = E * R
LG = ER + E
NT = 4096 // TM

_DN_T = (((1,), (1,)), ((), ()))
_DN_H = (((1,), (1,)), ((), ()))
_DN_C = (((0,), (0,)), ((), ()))


def _fused_kernel(x_ref, w_ref, b_ref, lg_ref, rf_ref, o_ref,
                  wb_ref, lgb_ref, rfb_ref, xb_ref, hw_ref):
    i = pl.program_id(0)

    @pl.when(i == 0)
    def _cast_small():
        lgb_ref[...] = lg_ref[...].astype(jnp.bfloat16)
        rfb_ref[...] = rf_ref[...].astype(jnp.bfloat16)

    @pl.when(i > 0)
    def _heavy():
        xp = xb_ref[...]
        out = jax.lax.dot_general(xp, wb_ref[...], _DN_T,
                                  preferred_element_type=jnp.float32)
        out += jax.lax.dot_general(hw_ref[...], rfb_ref[...], _DN_C,
                                   preferred_element_type=jnp.float32)
        o_ref[...] = out + b_ref[...]

    @pl.when(i < NT)
    def _gate():
        xb = x_ref[...].astype(jnp.bfloat16)
        xb_ref[...] = xb

        htg = jax.lax.dot_general(lgb_ref[...], xb, _DN_H,
                                  preferred_element_type=jnp.float32)
        h_t = htg[:ER, :]
        logits = htg[ER:, :]

        mx = jnp.max(logits, axis=0, keepdims=True)
        ex = jnp.exp(logits - mx)
        scores = ex / jnp.sum(ex, axis=0, keepdims=True)

        idx = jax.lax.broadcasted_iota(jnp.int32, scores.shape, 0)
        m1 = jnp.max(scores, axis=0, keepdims=True)
        i1 = jnp.min(jnp.where(scores == m1, idx, E), axis=0, keepdims=True)
        sel1 = idx == i1
        masked = jnp.where(sel1, -jnp.inf, scores)
        m2 = jnp.max(masked, axis=0, keepdims=True)
        i2 = jnp.min(jnp.where(masked == m2, idx, E), axis=0, keepdims=True)
        comb = jnp.where(sel1 | (idx == i2), scores, 0.0)

        comb_wide_t = jnp.repeat(comb, R, axis=0)
        hw_ref[...] = (h_t * (comb_wide_t * SCALING)).astype(jnp.bfloat16)

    @pl.when(i == 0)
    def _cast_w():
        wb_ref[...] = w_ref[...].astype(jnp.bfloat16)


def _x_map(i):
    return (jnp.minimum(i, NT - 1), 0)


def _o_map(i):
    return (jnp.maximum(i - 1, 0), 0)


@jax.jit
def _run(flat, w, b2, lg, rights_flat):
    T, D = flat.shape
    grid = (NT + 1,)
    return pl.pallas_call(
        _fused_kernel,
        grid=grid,
        in_specs=[
            pl.BlockSpec((TM, D), _x_map),
            pl.BlockSpec((D, D), lambda i: (0, 0)),
            pl.BlockSpec((1, D), lambda i: (0, 0)),
            pl.BlockSpec((LG, D), lambda i: (0, 0)),
            pl.BlockSpec((ER, D), lambda i: (0, 0)),
        ],
        out_specs=pl.BlockSpec((TM, D), _o_map),
        out_shape=jax.ShapeDtypeStruct((T, D), jnp.float32),
        scratch_shapes=[
            pltpu.VMEM((D, D), jnp.bfloat16),
            pltpu.VMEM((LG, D), jnp.bfloat16),
            pltpu.VMEM((ER, D), jnp.bfloat16),
            pltpu.VMEM((TM, D), jnp.bfloat16),
            pltpu.VMEM((ER, TM), jnp.bfloat16),
        ],
    )(flat, w, b2, lg, rights_flat)


def kernel(hidden_states, W_lin, b_lin, gate_w, lefts, rights):
    bsz, seq_len, dim = hidden_states.shape
    flat = hidden_states.reshape(-1, dim)
    d = lefts.shape[1]
    lefts_t = lefts.transpose(0, 2, 1).reshape(ER, d)
    lg = jnp.concatenate([lefts_t, gate_w], axis=0)
    rights_flat = rights.reshape(ER, -1)
    b2 = b_lin.reshape(1, -1)
    out = _run(flat, W_lin, b2, lg, rights_flat)
    return out.reshape(bsz, seq_len, -1)


# speedup vs baseline: 1.0215x; 1.0215x over previous
import jax
import jax.numpy as jnp
from jax.experimental import pallas as pl
from jax.experimental.pallas import tpu as pltpu

E = 8
K = 2
R = 16
ALPHA = 16
SCALING = ALPHA / R

TM = 512
ER = E * R
LG = ER + E

_DN_T = (((1,), (1,)), ((), ()))
_DN_H = (((1,), (1,)), ((), ()))
_DN_C = (((0,), (0,)), ((), ()))


def _fused_kernel(x_ref, w_ref, b_ref, lg_ref, rf_ref, o_ref,
                  wb_ref, lgb_ref, rfb_ref):
    i = pl.program_id(0)

    @pl.when(i == 0)
    def _cast_weights():
        wb_ref[...] = w_ref[...].astype(jnp.bfloat16)
        lgb_ref[...] = lg_ref[...].astype(jnp.bfloat16)
        rfb_ref[:ER, :] = rf_ref[...].astype(jnp.bfloat16)
        rfb_ref[ER:, :] = b_ref[...].astype(jnp.bfloat16)

    @pl.when(i > 0)
    def _compute():
        xb = x_ref[...].astype(jnp.bfloat16)

        out = jax.lax.dot_general(xb, wb_ref[...], _DN_T,
                                  preferred_element_type=jnp.float32)

        htg = jax.lax.dot_general(lgb_ref[...], xb, _DN_H,
                                  preferred_element_type=jnp.float32)
        h_t = htg[:ER, :]
        logits = htg[ER:, :]

        mx = jnp.max(logits, axis=0, keepdims=True)
        ex = jnp.exp(logits - mx)
        scores = ex / jnp.sum(ex, axis=0, keepdims=True)

        idx = jax.lax.broadcasted_iota(jnp.int32, scores.shape, 0)
        m1 = jnp.max(scores, axis=0, keepdims=True)
        i1 = jnp.min(jnp.where(scores == m1, idx, E), axis=0, keepdims=True)
        sel1 = idx == i1
        masked = jnp.where(sel1, -jnp.inf, scores)
        m2 = jnp.max(masked, axis=0, keepdims=True)
        i2 = jnp.min(jnp.where(masked == m2, idx, E), axis=0, keepdims=True)
        comb = jnp.where(sel1 | (idx == i2), scores, 0.0)

        comb_wide_t = jnp.repeat(comb, R, axis=0)

        hw_t = (h_t * (comb_wide_t * SCALING)).astype(jnp.bfloat16)
        ones = jnp.ones((1, TM), dtype=jnp.bfloat16)
        hw_aug = jnp.concatenate([hw_t, ones], axis=0)

        out += jax.lax.dot_general(hw_aug, rfb_ref[...], _DN_C,
                                   preferred_element_type=jnp.float32)
        o_ref[...] = out


def _x_map(i):
    j = jnp.maximum(i - 1, 0)
    return (j, 0)


@jax.jit
def _run(flat, w, b2, lg, rights_flat):
    T, D = flat.shape
    grid = (T // TM + 1,)
    return pl.pallas_call(
        _fused_kernel,
        grid=grid,
        in_specs=[
            pl.BlockSpec((TM, D), _x_map),
            pl.BlockSpec((D, D), lambda i: (0, 0)),
            pl.BlockSpec((1, D), lambda i: (0, 0)),
            pl.BlockSpec((LG, D), lambda i: (0, 0)),
            pl.BlockSpec((ER, D), lambda i: (0, 0)),
        ],
        out_specs=pl.BlockSpec((TM, D), _x_map),
        out_shape=jax.ShapeDtypeStruct((T, D), jnp.float32),
        scratch_shapes=[
            pltpu.VMEM((D, D), jnp.bfloat16),
            pltpu.VMEM((LG, D), jnp.bfloat16),
            pltpu.VMEM((ER + 1, D), jnp.bfloat16),
        ],
    )(flat, w, b2, lg, rights_flat)


def kernel(hidden_states, W_lin, b_lin, gate_w, lefts, rights):
    bsz, seq_len, dim = hidden_states.shape
    flat = hidden_states.reshape(-1, dim)
    d = lefts.shape[1]
    lefts_t = lefts.transpose(0, 2, 1).reshape(ER, d)
    lg = jnp.concatenate([lefts_t, gate_w], axis=0)
    rights_flat = rights.reshape(ER, -1)
    b2 = b_lin.reshape(1, -1)
    out = _run(flat, W_lin, b2, lg, rights_flat)
    return out.reshape(bsz, seq_len, -1)


# speedup vs baseline: 1.0237x; 1.0022x over previous
import jax
import jax.numpy as jnp
from jax.experimental import pallas as pl
from jax.experimental.pallas import tpu as pltpu

E = 8
K = 2
R = 16
ALPHA = 16
SCALING = ALPHA / R

TM = 512
ER = E * R
LG = ER + E
HK = 1024

_DN_T = (((1,), (1,)), ((), ()))
_DN_H = (((1,), (1,)), ((), ()))
_DN_C = (((0,), (0,)), ((), ()))


def _fused_kernel(x_ref, w_ref, b_ref, lg_ref, rf_ref, o_ref,
                  wb_ref, lgb_ref, rfb_ref):
    i = pl.program_id(0)

    @pl.when(i == 0)
    def _cast_weights():
        wb_ref[:, :HK] = w_ref[...].astype(jnp.bfloat16)
        lgb_ref[...] = lg_ref[...].astype(jnp.bfloat16)
        rfb_ref[:ER, :] = rf_ref[...].astype(jnp.bfloat16)
        rfb_ref[ER:, :] = b_ref[...].astype(jnp.bfloat16)

    @pl.when(i == 1)
    def _cast_w2():
        wb_ref[:, HK:] = w_ref[...].astype(jnp.bfloat16)

    @pl.when(i > 0)
    def _compute():
        xb = x_ref[...].astype(jnp.bfloat16)

        out = jax.lax.dot_general(xb[:, :HK], wb_ref[:, :HK], _DN_T,
                                  preferred_element_type=jnp.float32)
        out += jax.lax.dot_general(xb[:, HK:], wb_ref[:, HK:], _DN_T,
                                   preferred_element_type=jnp.float32)

        htg = jax.lax.dot_general(lgb_ref[...], xb, _DN_H,
                                  preferred_element_type=jnp.float32)
        h_t = htg[:ER, :]
        logits = htg[ER:, :]

        mx = jnp.max(logits, axis=0, keepdims=True)
        ex = jnp.exp(logits - mx)
        scores = ex / jnp.sum(ex, axis=0, keepdims=True)

        idx = jax.lax.broadcasted_iota(jnp.int32, scores.shape, 0)
        m1 = jnp.max(scores, axis=0, keepdims=True)
        i1 = jnp.min(jnp.where(scores == m1, idx, E), axis=0, keepdims=True)
        sel1 = idx == i1
        masked = jnp.where(sel1, -jnp.inf, scores)
        m2 = jnp.max(masked, axis=0, keepdims=True)
        i2 = jnp.min(jnp.where(masked == m2, idx, E), axis=0, keepdims=True)
        comb = jnp.where(sel1 | (idx == i2), scores, 0.0)

        comb_wide_t = jnp.repeat(comb, R, axis=0)

        hw_t = (h_t * (comb_wide_t * SCALING)).astype(jnp.bfloat16)
        ones = jnp.ones((1, TM), dtype=jnp.bfloat16)
        hw_aug = jnp.concatenate([hw_t, ones], axis=0)

        out += jax.lax.dot_general(hw_aug, rfb_ref[...], _DN_C,
                                   preferred_element_type=jnp.float32)
        o_ref[...] = out


def _x_map(i):
    j = jnp.maximum(i - 1, 0)
    return (j, 0)


@jax.jit
def _run(flat, w, b2, lg, rights_flat):
    T, D = flat.shape
    grid = (T // TM + 1,)
    return pl.pallas_call(
        _fused_kernel,
        grid=grid,
        in_specs=[
            pl.BlockSpec((TM, D), _x_map),
            pl.BlockSpec((D, HK), lambda i: (0, jnp.minimum(i, 1))),
            pl.BlockSpec((1, D), lambda i: (0, 0)),
            pl.BlockSpec((LG, D), lambda i: (0, 0)),
            pl.BlockSpec((ER, D), lambda i: (0, 0)),
        ],
        out_specs=pl.BlockSpec((TM, D), _x_map),
        out_shape=jax.ShapeDtypeStruct((T, D), jnp.float32),
        scratch_shapes=[
            pltpu.VMEM((D, D), jnp.bfloat16),
            pltpu.VMEM((LG, D), jnp.bfloat16),
            pltpu.VMEM((ER + 1, D), jnp.bfloat16),
        ],
    )(flat, w, b2, lg, rights_flat)


def kernel(hidden_states, W_lin, b_lin, gate_w, lefts, rights):
    bsz, seq_len, dim = hidden_states.shape
    flat = hidden_states.reshape(-1, dim)
    d = lefts.shape[1]
    lefts_t = lefts.transpose(0, 2, 1).reshape(ER, d)
    lg = jnp.concatenate([lefts_t, gate_w], axis=0)
    rights_flat = rights.reshape(ER, -1)
    b2 = b_lin.reshape(1, -1)
    out = _run(flat, W_lin, b2, lg, rights_flat)
    return out.reshape(bsz, seq_len, -1)
